# baseline (speedup 1.0000x reference)
import functools

import jax
import jax.numpy as jnp
from jax import lax
from jax.experimental import pallas as pl
from jax.experimental.pallas import tpu as pltpu
from jax.experimental.pallas import tpu_sc as plsc

D = 128
NC, NS = 2, 16
NW = NC * NS
GE = 2
NBUF = 4


@functools.partial(jax.jit, static_argnames=("batch", "hist"))
def _lookup(idxp, table, *, batch, hist):
    gl = GE * hist
    glp = (gl + 7) // 8 * 8
    groups = batch // (NW * GE)
    epw = batch // NW
    mesh = plsc.VectorSubcoreMesh(core_axis_name="c", subcore_axis_name="s")

    @functools.partial(
        pl.kernel,
        out_type=jax.ShapeDtypeStruct((batch, hist, D), jnp.float32),
        mesh=mesh,
        scratch_types=[
            pltpu.VMEM((groups * glp,), jnp.int32),
            pltpu.VMEM((NBUF, glp, D), jnp.float32),
            pltpu.SemaphoreType.DMA((NBUF,)),
            pltpu.SemaphoreType.DMA((NBUF,)),
        ],
    )
    def body(table_hbm, idx_hbm, out_hbm, idx_v, rows_v, gsem, wsem):
        wid = lax.axis_index("s") * NC + lax.axis_index("c")
        pltpu.sync_copy(idx_hbm.at[pl.ds(wid * groups * glp, groups * glp)],
                        idx_v)
        ebase = wid * epw

        def fire_gather(g, b):
            pltpu.async_copy(table_hbm.at[idx_v.at[pl.ds(g * glp, glp)]],
                             rows_v.at[b], gsem.at[b])

        def wait_gather(b):
            pltpu.make_async_copy(table_hbm.at[idx_v.at[pl.ds(0, glp)]],
                                  rows_v.at[b], gsem.at[b]).wait()

        def fire_writes(g, b):
            for e in range(GE):
                pltpu.async_copy(rows_v.at[b].at[pl.ds(e * hist, hist)],
                                 out_hbm.at[ebase + g * GE + e], wsem.at[b])

        def wait_writes(b):
            for e in range(GE):
                pltpu.make_async_copy(rows_v.at[b].at[pl.ds(0, hist)],
                                      out_hbm.at[0], wsem.at[b]).wait()

        for b in range(NBUF):
            fire_gather(b, b)

        @pl.loop(0, groups - NBUF, step=NBUF)
        def _(g0):
            for b in range(NBUF):
                wait_gather(b)
                fire_writes(g0 + b, b)
            for b in range(NBUF):
                wait_writes(b)
                fire_gather(g0 + NBUF + b, b)

        for b in range(NBUF):
            wait_gather(b)
            fire_writes(groups - NBUF + b, b)
        for b in range(NBUF):
            wait_writes(b)

    return body(table, idxp)


def kernel(input_ids, word_embeddings):
    batch, hist = input_ids.shape
    gl = GE * hist
    glp = (gl + 7) // 8 * 8
    idx = input_ids.astype(jnp.int32).reshape(batch // GE, gl)
    idxp = jnp.pad(idx, ((0, 0), (0, glp - gl))).reshape(-1)
    return _lookup(idxp, word_embeddings, batch=batch, hist=hist)
